# baseline (speedup 1.0000x reference)
import functools
import math

import jax
import jax.numpy as jnp
from jax import lax
from jax.experimental import pallas as pl
from jax.experimental.pallas import tpu as pltpu
from jax.experimental.pallas import tpu_sc as plsc

_SCALE = 64.0
_MARGIN = 0.5
_COS_M = math.cos(_MARGIN)
_SIN_M = math.sin(_MARGIN)
_THRESH = -math.cos(_MARGIN)
_MONO = math.sin(_MARGIN) * _MARGIN
_NEG = -1e30


def _rowsum_kernel(x_ref, s_ref, acc, *, BC, C, CB):
    c = pl.program_id(1)

    @pl.when(c == 0)
    def _init():
        acc[...] = jnp.zeros_like(acc)

    x = x_ref[...]

    def _tail():
        col = lax.broadcasted_iota(jnp.int32, x.shape, 1)
        xs = jnp.where(col < C - c * BC, x * _SCALE - _SCALE, _NEG)
        return jnp.sum(jnp.exp(xs), axis=1, keepdims=True)

    def _full():
        return jnp.sum(jnp.exp(x * _SCALE - _SCALE), axis=1, keepdims=True)

    acc[...] += lax.cond(c == CB - 1, _tail, _full)

    @pl.when(c == CB - 1)
    def _out():
        s_ref[...] = acc[...]


def _combine_kernel(s_ref, t_ref, out_ref, *, B):
    s = s_ref[...]
    t = t_ref[...]
    tr = t * _COS_M - _SIN_M * jnp.sqrt(jnp.maximum(1.0 - t * t, 0.0))
    tr = jnp.where(t > _THRESH, tr, t - _MONO)
    e_t = jnp.exp(t * _SCALE - _SCALE)
    e_tr = jnp.exp(tr * _SCALE - _SCALE)
    lse = jnp.log(s - e_t + e_tr) + _SCALE
    out_ref[...] = jnp.sum(lse - _SCALE * tr).reshape(1, 1) / B


def _sc_gather(table_flat, flat_idx, B):
    info = plsc.get_sparse_core_info()
    nw = info.num_cores * info.num_subcores
    b_per_w = B // nw
    mesh = plsc.VectorSubcoreMesh(core_axis_name="c", subcore_axis_name="s")

    @functools.partial(
        pl.kernel,
        mesh=mesh,
        out_type=jax.ShapeDtypeStruct((B,), jnp.float32),
        scratch_types=[
            pltpu.VMEM((b_per_w,), jnp.int32),
            pltpu.VMEM((b_per_w,), jnp.float32),
            pltpu.SemaphoreType.DMA,
        ],
    )
    def gather_k(table_hbm, idx_hbm, out_hbm, idx_v, vals_v, sem):
        wid = lax.axis_index("s") * info.num_cores + lax.axis_index("c")
        base = wid * b_per_w
        pltpu.sync_copy(idx_hbm.at[pl.ds(base, b_per_w)], idx_v)
        pltpu.async_copy(table_hbm.at[idx_v], vals_v, sem).wait()
        pltpu.sync_copy(vals_v, out_hbm.at[pl.ds(base, b_per_w)])

    return gather_k(table_flat, flat_idx)


@jax.jit
def kernel(cosine, label):
    B, C = cosine.shape
    BR, BC = 256, 4096
    R = B // BR
    CB = pl.cdiv(C, BC)

    flat_idx = jnp.arange(B, dtype=jnp.int32) * C + label.astype(jnp.int32)
    tgt = _sc_gather(jnp.reshape(cosine, (B * C,)), flat_idx, B)

    s = pl.pallas_call(
        functools.partial(_rowsum_kernel, BC=BC, C=C, CB=CB),
        grid=(R, CB),
        in_specs=[pl.BlockSpec((BR, BC), lambda r, c: (r, c))],
        out_specs=pl.BlockSpec((BR, 1), lambda r, c: (r, 0)),
        out_shape=jax.ShapeDtypeStruct((B, 1), jnp.float32),
        scratch_shapes=[pltpu.VMEM((BR, 1), jnp.float32)],
    )(cosine)

    out = pl.pallas_call(
        functools.partial(_combine_kernel, B=B),
        grid=(1,),
        in_specs=[
            pl.BlockSpec((B, 1), lambda i: (0, 0)),
            pl.BlockSpec((B, 1), lambda i: (0, 0)),
        ],
        out_specs=pl.BlockSpec((1, 1), lambda i: (0, 0)),
        out_shape=jax.ShapeDtypeStruct((1, 1), jnp.float32),
    )(s, tgt.reshape(B, 1))
    return out[0, 0]


# speedup vs baseline: 1.9959x; 1.9959x over previous
import functools
import math

import jax
import jax.numpy as jnp
from jax import lax
from jax.experimental import pallas as pl
from jax.experimental.pallas import tpu as pltpu

_SCALE = 64.0
_MARGIN = 0.5
_COS_M = math.cos(_MARGIN)
_SIN_M = math.sin(_MARGIN)
_THRESH = -math.cos(_MARGIN)
_MONO = math.sin(_MARGIN) * _MARGIN
_NEG = -1e30


def _arc_kernel(lab_ref, x_ref, out_ref, s_s, t_s, *, BC, C, CB, R, B):
    r = pl.program_id(0)
    c = pl.program_id(1)

    @pl.when(c == 0)
    def _init():
        s_s[...] = jnp.zeros_like(s_s)
        t_s[...] = jnp.zeros_like(t_s)

    x = x_ref[...]
    lab = lab_ref[0]
    rel = lab - c * BC
    col = lax.broadcasted_iota(jnp.int32, x.shape, 1)
    sub = col == rel
    hit = (rel >= 0) & (rel < BC)

    t = jnp.sum(jnp.where(sub, x, 0.0), axis=1, keepdims=True)
    tr = t * _COS_M - _SIN_M * jnp.sqrt(jnp.maximum(1.0 - t * t, 0.0))
    tr = jnp.where(t > _THRESH, tr, t - _MONO)
    tr_scaled = _SCALE * tr
    t_s[...] = jnp.where(hit, tr_scaled, t_s[...])

    xs = jnp.where(sub, tr_scaled - _SCALE, x * _SCALE - _SCALE)

    def _tail():
        return jnp.sum(jnp.exp(jnp.where(col < C - c * BC, xs, _NEG)),
                       axis=1, keepdims=True)

    def _full():
        return jnp.sum(jnp.exp(xs), axis=1, keepdims=True)

    s_s[...] += lax.cond(c == CB - 1, _tail, _full)

    @pl.when(c == CB - 1)
    def _finish():
        lse = jnp.log(s_s[...]) + _SCALE
        part = jnp.sum(lse - t_s[...]).reshape(1, 1)

        @pl.when(r == 0)
        def _zero():
            out_ref[...] = jnp.zeros_like(out_ref)

        out_ref[...] += part

        @pl.when(r == R - 1)
        def _mean():
            out_ref[...] = out_ref[...] / B


def _build_call(B, C, BR, BC):
    R = B // BR
    CB = pl.cdiv(C, BC)
    return pl.pallas_call(
        functools.partial(_arc_kernel, BC=BC, C=C, CB=CB, R=R, B=B),
        grid=(R, CB),
        in_specs=[
            pl.BlockSpec((1, BR, 1), lambda r, c: (r, 0, 0)),
            pl.BlockSpec((BR, BC), lambda r, c: (r, c)),
        ],
        out_specs=pl.BlockSpec((1, 1), lambda r, c: (0, 0)),
        out_shape=jax.ShapeDtypeStruct((1, 1), jnp.float32),
        scratch_shapes=[
            pltpu.VMEM((BR, 1), jnp.float32),
            pltpu.VMEM((BR, 1), jnp.float32),
        ],
    )


@jax.jit
def kernel(cosine, label):
    B, C = cosine.shape
    BR, BC = 256, 4096
    lab3 = label.astype(jnp.int32).reshape(B // BR, BR, 1)
    out = _build_call(B, C, BR, BC)(lab3, cosine)
    return out[0, 0]
